# baseline (speedup 1.0000x reference)
import functools

import jax
import jax.numpy as jnp
from jax import lax
from jax.experimental import pallas as pl
from jax.experimental.pallas import tpu as pltpu
from jax.experimental.pallas import tpu_sc as plsc

_NC = 2
_NS = 16
_NW = _NC * _NS
_D = 256
_CHUNK = 128
_NBUF = 3


@functools.cache
def _make_gather(out_shape, B):
    b_per_w = B // _NW
    n_chunks = b_per_w // _CHUNK
    mesh = plsc.VectorSubcoreMesh(core_axis_name="c", subcore_axis_name="s")
    n_outer = -(-n_chunks // _NBUF)

    @functools.partial(
        pl.kernel,
        mesh=mesh,
        out_type=jax.ShapeDtypeStruct(out_shape, jnp.float32),
        scratch_types=[
            pltpu.VMEM((b_per_w,), jnp.int32),
            *[pltpu.VMEM((_CHUNK, _D), jnp.float32) for _ in range(_NBUF)],
            *[pltpu.SemaphoreType.DMA for _ in range(2 * _NBUF)],
        ],
    )
    def k(idx_hbm, table_hbm, out_hbm, idx_v, *bufs_sems):
        bufs = bufs_sems[:_NBUF]
        gsems = bufs_sems[_NBUF:2 * _NBUF]
        osems = bufs_sems[2 * _NBUF:]
        wid = lax.axis_index("s") * _NC + lax.axis_index("c")
        base = wid * b_per_w
        out_flat = out_hbm.reshape(B, _D)
        pltpu.sync_copy(idx_hbm.at[pl.ds(wid * b_per_w, b_per_w)], idx_v)

        def gather(j, p):
            idx = idx_v.at[pl.ds(j * _CHUNK, _CHUNK)]
            pltpu.async_copy(table_hbm.at[idx], bufs[p], gsems[p])

        def gather_wait(j, p):
            idx = idx_v.at[pl.ds(j * _CHUNK, _CHUNK)]
            pltpu.make_async_copy(table_hbm.at[idx], bufs[p], gsems[p]).wait()

        def put(j, p):
            dst = out_flat.at[pl.ds(base + j * _CHUNK, _CHUNK)]
            pltpu.async_copy(bufs[p], dst, osems[p])

        def put_wait(j, p):
            dst = out_flat.at[pl.ds(base + j * _CHUNK, _CHUNK)]
            pltpu.make_async_copy(bufs[p], dst, osems[p]).wait()

        gather(0, 0)
        gather(1, 1)

        def body(jj, carry):
            for p in range(_NBUF):
                j = jj * _NBUF + p
                pprev = (p - 1) % _NBUF

                @pl.when(j < n_chunks)
                def _():
                    gather_wait(j, p)
                    put(j, p)

                @pl.when(jnp.logical_and(j >= 1, j < n_chunks))
                def _():
                    put_wait(j - 1, pprev)

                @pl.when(j + 2 < n_chunks)
                def _():
                    gather(j + 2, pprev)

            return carry

        lax.fori_loop(0, n_outer, body, 0)
        put_wait(n_chunks - 1, (n_chunks - 1) % _NBUF)

    return k


def kernel(word_tokens, table):
    B = word_tokens.size
    idx = word_tokens if word_tokens.dtype == jnp.int32 else word_tokens.astype(jnp.int32)
    idx = idx.reshape(B)
    return _make_gather((*word_tokens.shape, _D), B)(idx, table)
